# baseline (speedup 1.0000x reference)
import functools

import jax
import jax.numpy as jnp
from jax import lax
from jax.experimental import pallas as pl
from jax.experimental.pallas import tpu as pltpu
from jax.experimental.pallas import tpu_sc as plsc

VOCAB = 1000
BATCH = 4096
SEQ = 20
NC, NS = 2, 16
NW = NC * NS
NVT = VOCAB // 8
VT_PER_W = 4
NBT = BATCH // 128
NGRP = BATCH // 16


def _sc_bigram(table_t, idx_t):
    mesh = plsc.VectorSubcoreMesh(core_axis_name="c", subcore_axis_name="s")

    @functools.partial(
        pl.kernel,
        mesh=mesh,
        compiler_params=pltpu.CompilerParams(use_tc_tiling_on_sc=False,
                                             needs_layout_passes=False),
        out_type=jax.ShapeDtypeStruct((SEQ, NVT, NBT, 8, 128), jnp.float32),
        scratch_types=[
            pltpu.VMEM((VT_PER_W, 8, VOCAB), jnp.float32),
            pltpu.VMEM((2, BATCH), jnp.int32),
            pltpu.VMEM((NBT, 8, 128), jnp.float32),
            pltpu.VMEM((NBT, 8, 128), jnp.float32),
            pltpu.SemaphoreType.DMA,
            pltpu.SemaphoreType.DMA,
            pltpu.SemaphoreType.DMA,
        ],
    )
    def k(tab_hbm, idx_hbm, out_hbm, slab_v, idx_v, ob0, ob1, o0, o1, isem):
        obufs = (ob0, ob1)
        osems = (o0, o1)
        sid = lax.axis_index("s")
        wid = sid * NC + lax.axis_index("c")
        vr0 = wid * VT_PER_W
        nvr = jnp.minimum(NVT - vr0, VT_PER_W)

        for j in range(VT_PER_W):
            @pl.when(j < nvr)
            def _():
                pltpu.sync_copy(tab_hbm.at[pl.ds((vr0 + j) * 8, 8)],
                                slab_v.at[j])

        v8s = [jnp.full((16,), v8, jnp.int32) for v8 in range(8)]

        def fill(jv, sm2, half):
            @plsc.parallel_loop(0, NGRP, unroll=8)
            def _(g):
                vidx = idx_v[sm2, pl.ds(g * 16, 16)]
                c = g // 8
                lb = (g % 8) * 16
                for v8 in range(8):
                    val = plsc.load_gather(slab_v, [jv, v8s[v8], vidx])
                    obufs[half][c, v8, pl.ds(lb, 16)] = val

        def ocopy_desc(s, vr, half):
            return pltpu.make_async_copy(obufs[half], out_hbm.at[s, vr],
                                         osems[half])

        pltpu.sync_copy(idx_hbm.at[0], idx_v.at[0])

        def idx_desc(s, sm2):
            return pltpu.make_async_copy(idx_hbm.at[s], idx_v.at[sm2], isem)

        def pair_body(tp, _):
            for half in range(2):
                t = 2 * tp + half
                s = t // nvr
                j = t % nvr
                sm2 = s % 2

                @pl.when(jnp.logical_and(j == 0, s > 0))
                def _():
                    idx_desc(s, sm2).wait()

                @pl.when(jnp.logical_and(j == nvr - 1, s < SEQ - 1))
                def _():
                    idx_desc(s + 1, (s + 1) % 2).start()

                @pl.when(tp > 0)
                def _():
                    ocopy_desc(0, 0, half).wait()

                jv = jnp.full((16,), j, jnp.int32)
                fill(jv, sm2, half)
                ocopy_desc(s, vr0 + j, half).start()
            return _

        lax.fori_loop(0, SEQ * nvr // 2, pair_body, None)
        for half in range(2):
            ocopy_desc(0, 0, half).wait()

    return k(table_t, idx_t)


@jax.jit
def kernel(idx, logits_table):
    table_t = logits_table.T
    idx_t = idx.astype(jnp.int32).T
    out5 = _sc_bigram(table_t, idx_t)
    return out5.transpose(2, 4, 0, 1, 3).reshape(BATCH, SEQ, VOCAB)


# speedup vs baseline: 1.0074x; 1.0074x over previous
import functools

import jax
import jax.numpy as jnp
from jax import lax
from jax.experimental import pallas as pl
from jax.experimental.pallas import tpu as pltpu
from jax.experimental.pallas import tpu_sc as plsc

VOCAB = 1000
BATCH = 4096
SEQ = 20
NC, NS = 2, 16
NW = NC * NS
NVT = VOCAB // 8
VT_PER_W = 4
NBT = BATCH // 128
NGRP = BATCH // 16


def _sc_bigram(table_t, idx_t):
    mesh = plsc.VectorSubcoreMesh(core_axis_name="c", subcore_axis_name="s")

    @functools.partial(
        pl.kernel,
        mesh=mesh,
        compiler_params=pltpu.CompilerParams(use_tc_tiling_on_sc=False,
                                             needs_layout_passes=False),
        out_type=jax.ShapeDtypeStruct((SEQ, NVT, NBT, 8, 128), jnp.float32),
        scratch_types=[
            pltpu.VMEM((VT_PER_W, 8, VOCAB), jnp.float32),
            pltpu.VMEM((2, BATCH), jnp.int32),
            pltpu.VMEM((NBT, 8, 128), jnp.float32),
            pltpu.VMEM((NBT, 8, 128), jnp.float32),
            pltpu.SemaphoreType.DMA,
            pltpu.SemaphoreType.DMA,
            pltpu.SemaphoreType.DMA,
        ],
    )
    def k(tab_hbm, idx_hbm, out_hbm, slab_v, idx_v, ob0, ob1, o0, o1, isem):
        obufs = (ob0, ob1)
        osems = (o0, o1)
        sid = lax.axis_index("s")
        wid = sid * NC + lax.axis_index("c")
        vr0 = wid * VT_PER_W
        nvr = jnp.minimum(NVT - vr0, VT_PER_W)

        for j in range(VT_PER_W):
            @pl.when(j < nvr)
            def _():
                pltpu.sync_copy(tab_hbm.at[pl.ds((vr0 + j) * 8, 8)],
                                slab_v.at[j])

        v8s = [jnp.full((16,), v8, jnp.int32) for v8 in range(8)]

        def fill(jv, sm2, half):
            @plsc.parallel_loop(0, NGRP, unroll=4)
            def _(g):
                vidx = idx_v[sm2, pl.ds(g * 16, 16)]
                c = g // 8
                lb = (g % 8) * 16
                for v8 in range(8):
                    val = plsc.load_gather(slab_v, [jv, v8s[v8], vidx])
                    obufs[half][c, v8, pl.ds(lb, 16)] = val

        def ocopy_desc(s, vr, half):
            return pltpu.make_async_copy(obufs[half], out_hbm.at[s, vr],
                                         osems[half])

        pltpu.sync_copy(idx_hbm.at[0], idx_v.at[0])

        def idx_desc(s, sm2):
            return pltpu.make_async_copy(idx_hbm.at[s], idx_v.at[sm2], isem)

        def pair_body(tp, _):
            for half in range(2):
                t = 2 * tp + half
                s = t // nvr
                j = t % nvr
                sm2 = s % 2

                @pl.when(jnp.logical_and(j == 0, s > 0))
                def _():
                    idx_desc(s, sm2).wait()

                @pl.when(jnp.logical_and(j == nvr - 1, s < SEQ - 1))
                def _():
                    idx_desc(s + 1, (s + 1) % 2).start()

                @pl.when(tp > 0)
                def _():
                    ocopy_desc(0, 0, half).wait()

                jv = jnp.full((16,), j, jnp.int32)
                fill(jv, sm2, half)
                ocopy_desc(s, vr0 + j, half).start()
            return _

        lax.fori_loop(0, SEQ * nvr // 2, pair_body, None)
        for half in range(2):
            ocopy_desc(0, 0, half).wait()

    return k(table_t, idx_t)


@jax.jit
def kernel(idx, logits_table):
    table_t = logits_table.T
    idx_t = idx.astype(jnp.int32).T
    out5 = _sc_bigram(table_t, idx_t)
    return out5.transpose(2, 4, 0, 1, 3).reshape(BATCH, SEQ, VOCAB)
